# baseline (speedup 1.0000x reference)
import functools
import math

import jax
import jax.numpy as jnp
from jax import lax
from jax.experimental import pallas as pl
from jax.experimental.pallas import tpu as pltpu
from jax.experimental.pallas import tpu_sc as plsc

VOCAB = 100000
EMB = 64
SCALE = math.sqrt(EMB)

BATCH = 4096
SEQ = 200
B = BATCH * SEQ
NC, NS = 2, 16
NW = NC * NS
B_PER_W = B // NW
BLK = 128
NBLK_W = B_PER_W // BLK
NBT = BATCH // BLK


_mesh = plsc.VectorSubcoreMesh(core_axis_name="c", subcore_axis_name="s")


def _transpose_block(g, gt):
    lanes = lax.iota(jnp.int32, 16)
    rj = [lanes + j * 16 for j in range(EMB // 16)]

    def row(i, cidx):
        for j in range(EMB // 16):
            v = g[i, pl.ds(j * 16, 16)] * SCALE
            plsc.store_scatter(gt, [rj[j], cidx], v)
        return cidx + 1

    lax.fori_loop(0, BLK, row, lanes * 0, unroll=2)


@functools.partial(
    pl.kernel,
    mesh=_mesh,
    out_type=jax.ShapeDtypeStruct((SEQ, 8, NBT, 8, BLK), jnp.float32),
    scratch_types=[
        pltpu.VMEM((B_PER_W,), jnp.int32),
        pltpu.VMEM((BLK, EMB), jnp.float32),
        pltpu.VMEM((BLK, EMB), jnp.float32),
        pltpu.VMEM((EMB, BLK + 1), jnp.float32),
        pltpu.VMEM((EMB, BLK + 1), jnp.float32),
        pltpu.SemaphoreType.DMA,
        pltpu.SemaphoreType.DMA,
        pltpu.SemaphoreType.DMA,
        pltpu.SemaphoreType.DMA,
    ],
    compiler_params=pltpu.CompilerParams(use_tc_tiling_on_sc=False, needs_layout_passes=False),
)
def _gather_t(idx_hbm, table_hbm, out_hbm, idx_v, g0, g1, gt0, gt1, sem0, sem1, semo0, semo1):
    wid = lax.axis_index("s") * NC + lax.axis_index("c")
    base = wid * B_PER_W
    blk0 = wid * NBLK_W

    pltpu.sync_copy(idx_hbm.at[pl.ds(base, B_PER_W)], idx_v)

    def idx_at(k):
        return idx_v.at[pl.ds(k * BLK, BLK)]

    def emit(k, g, gt, semo):
        bid = blk0 + k
        s = bid // NBT
        bt = bid % NBT
        _transpose_block(g, gt)
        for eh in range(8):
            pltpu.async_copy(gt.at[pl.ds(eh * 8, 8), pl.ds(0, BLK)],
                             out_hbm.at[s, eh, bt], semo)

    def drain_emit(k, gt, semo):
        bid = blk0 + k
        s = bid // NBT
        bt = bid % NBT
        for eh in range(8):
            pltpu.make_async_copy(gt.at[pl.ds(eh * 8, 8), pl.ds(0, BLK)],
                                  out_hbm.at[s, eh, bt], semo).wait()

    pltpu.async_copy(table_hbm.at[idx_at(0)], g0, sem0)

    def pair(p, carry):
        k0 = 2 * p
        k1 = k0 + 1
        pltpu.async_copy(table_hbm.at[idx_at(k1)], g1, sem1)
        pltpu.make_async_copy(table_hbm.at[idx_at(k0)], g0, sem0).wait()

        @pl.when(p > 0)
        def _():
            drain_emit(k0 - 2, gt0, semo0)

        emit(k0, g0, gt0, semo0)

        @pl.when(p < NBLK_W // 2 - 1)
        def _():
            pltpu.async_copy(table_hbm.at[idx_at(k0 + 2)], g0, sem0)

        pltpu.make_async_copy(table_hbm.at[idx_at(k1)], g1, sem1).wait()

        @pl.when(p > 0)
        def _():
            drain_emit(k1 - 2, gt1, semo1)

        emit(k1, g1, gt1, semo1)
        return carry

    lax.fori_loop(0, NBLK_W // 2, pair, 0)
    drain_emit(NBLK_W - 2, gt0, semo0)
    drain_emit(NBLK_W - 1, gt1, semo1)


def kernel(tokens, emb_weight):
    flat = tokens.T.reshape(-1).astype(jnp.int32)
    y = _gather_t(flat, emb_weight)
    return y.transpose(2, 4, 0, 1, 3).reshape(BATCH, SEQ, EMB)


# speedup vs baseline: 1.1432x; 1.1432x over previous
import functools
import math

import jax
import jax.numpy as jnp
from jax import lax
from jax.experimental import pallas as pl
from jax.experimental.pallas import tpu as pltpu
from jax.experimental.pallas import tpu_sc as plsc

VOCAB = 100000
EMB = 64
SCALE = math.sqrt(EMB)

BATCH = 4096
SEQ = 200
B = BATCH * SEQ
NC, NS = 2, 16
NW = NC * NS
B_PER_W = B // NW
BLK = 128
NBLK_W = B_PER_W // BLK
NBT = BATCH // BLK


def _scale_body(w_ref, o_ref):
    o_ref[...] = w_ref[...] * SCALE


_scale_table = pl.pallas_call(
    _scale_body,
    grid=(10,),
    in_specs=[pl.BlockSpec((VOCAB // 10, EMB), lambda i: (i, 0))],
    out_specs=pl.BlockSpec((VOCAB // 10, EMB), lambda i: (i, 0)),
    out_shape=jax.ShapeDtypeStruct((VOCAB, EMB), jnp.float32),
)

_mesh = plsc.VectorSubcoreMesh(core_axis_name="c", subcore_axis_name="s")


def _transpose_block(g, gt):
    lanes = lax.iota(jnp.int32, 16)
    rj = [lanes + j * 16 for j in range(EMB // 16)]

    def row(i, cidx):
        for j in range(EMB // 16):
            v = g[i, pl.ds(j * 16, 16)]
            plsc.store_scatter(gt, [rj[j], cidx], v)
        return cidx + 1

    lax.fori_loop(0, BLK, row, lanes * 0, unroll=4)


@functools.partial(
    pl.kernel,
    mesh=_mesh,
    out_type=jax.ShapeDtypeStruct((SEQ, 8, NBT, 8, BLK), jnp.float32),
    scratch_types=[
        pltpu.VMEM((B_PER_W,), jnp.int32),
        pltpu.VMEM((BLK, EMB), jnp.float32),
        pltpu.VMEM((BLK, EMB), jnp.float32),
        pltpu.VMEM((EMB, BLK + 1), jnp.float32),
        pltpu.VMEM((EMB, BLK + 1), jnp.float32),
        pltpu.SemaphoreType.DMA,
        pltpu.SemaphoreType.DMA,
        pltpu.SemaphoreType.DMA,
        pltpu.SemaphoreType.DMA,
    ],
    compiler_params=pltpu.CompilerParams(use_tc_tiling_on_sc=False, needs_layout_passes=False),
)
def _gather_t(idx_hbm, table_hbm, out_hbm, idx_v, g0, g1, gt0, gt1, sem0, sem1, semo0, semo1):
    wid = lax.axis_index("s") * NC + lax.axis_index("c")
    base = wid * B_PER_W
    blk0 = wid * NBLK_W

    pltpu.sync_copy(idx_hbm.at[pl.ds(base, B_PER_W)], idx_v)

    def idx_at(k):
        return idx_v.at[pl.ds(k * BLK, BLK)]

    def emit(k, g, gt, semo):
        bid = blk0 + k
        s = bid // NBT
        bt = bid % NBT
        _transpose_block(g, gt)
        for eh in range(8):
            pltpu.async_copy(gt.at[pl.ds(eh * 8, 8), pl.ds(0, BLK)],
                             out_hbm.at[s, eh, bt], semo)

    def drain_emit(k, gt, semo):
        bid = blk0 + k
        s = bid // NBT
        bt = bid % NBT
        for eh in range(8):
            pltpu.make_async_copy(gt.at[pl.ds(eh * 8, 8), pl.ds(0, BLK)],
                                  out_hbm.at[s, eh, bt], semo).wait()

    pltpu.async_copy(table_hbm.at[idx_at(0)], g0, sem0)

    def pair(p, carry):
        k0 = 2 * p
        k1 = k0 + 1
        pltpu.async_copy(table_hbm.at[idx_at(k1)], g1, sem1)
        pltpu.make_async_copy(table_hbm.at[idx_at(k0)], g0, sem0).wait()

        @pl.when(p > 0)
        def _():
            drain_emit(k0 - 2, gt0, semo0)

        emit(k0, g0, gt0, semo0)

        @pl.when(p < NBLK_W // 2 - 1)
        def _():
            pltpu.async_copy(table_hbm.at[idx_at(k0 + 2)], g0, sem0)

        pltpu.make_async_copy(table_hbm.at[idx_at(k1)], g1, sem1).wait()

        @pl.when(p > 0)
        def _():
            drain_emit(k1 - 2, gt1, semo1)

        emit(k1, g1, gt1, semo1)
        return carry

    lax.fori_loop(0, NBLK_W // 2, pair, 0)
    drain_emit(NBLK_W - 2, gt0, semo0)
    drain_emit(NBLK_W - 1, gt1, semo1)


def kernel(tokens, emb_weight):
    table = _scale_table(emb_weight)
    flat = tokens.T.reshape(-1).astype(jnp.int32)
    y = _gather_t(flat, table)
    return y.transpose(2, 4, 0, 1, 3).reshape(BATCH, SEQ, EMB)
